# baseline (speedup 1.0000x reference)
import functools

import jax
import jax.numpy as jnp
from jax import lax
from jax.experimental import pallas as pl
from jax.experimental.pallas import tpu as pltpu
from jax.experimental.pallas import tpu_sc as plsc

NNODE = 10000
D = 128
W = 144
NC, NS = 2, 16
NT = NC * NS
CHUNK = 32
K = 320
NBUF = 4
EPAD = NT * K * CHUNK
NZROW = 16
RPT = NNODE // NS

REL1 = ["p2b", "s2b", "b2b", "i2s", "p2s", "b2s", "b2p", "p2p", "s2p"]
SRC1 = ["p", "s", "b", "i", "p", "b", "b", "p", "s"]
GRP_DST = ["b", "s", "p"]
TID = {"i": 0, "s": 1, "p": 2, "b": 3}
REL2 = ["p2b", "s2b", "b2b"]


def _sc_segment_sums(nrel):
  mesh = plsc.VectorSubcoreMesh(
      core_axis_name="c", subcore_axis_name="s", num_cores=NC, num_subcores=NS)

  def body(table, src, dst, out, acc, r0, r1, r2, r3, sidx, didx, *sems):
    rows = [r0, r1, r2, r3]
    cid = lax.axis_index("c")
    sid = lax.axis_index("s")
    wid = cid * NS + sid
    base = sid * RPT

    def zero_rows():
      zvec = jnp.zeros((16,), jnp.float32)

      def zstore(i, _):
        rr = i // (W // 16)
        cc = (i % (W // 16)) * 16
        rows[0][rr, pl.ds(cc, 16)] = zvec
        rows[1][rr, pl.ds(cc, 16)] = zvec
        return _

      lax.fori_loop(0, CHUNK * (W // 16), zstore, 0)

    def zero_slice():
      def zcopy(t, _):
        pltpu.sync_copy(rows[0], acc.at[pl.ds(base + 2 * t * CHUNK, CHUNK)])
        pltpu.sync_copy(rows[1], acc.at[pl.ds(base + (2 * t + 1) * CHUNK, CHUNK)])
        return _

      lax.fori_loop(0, RPT // (2 * CHUNK), zcopy, 0)
      rem = RPT % (2 * CHUNK)
      if rem:
        pltpu.sync_copy(rows[0].at[pl.ds(0, rem)],
                        acc.at[pl.ds(base + (RPT // (2 * CHUNK)) * 2 * CHUNK, rem)])

    zero_rows()
    zero_slice()
    plsc.subcore_barrier()

    for r in range(nrel):
      pltpu.sync_copy(src.at[r, wid], sidx)
      pltpu.sync_copy(dst.at[r, wid], didx)
      for b in range(NBUF - 1):
        pltpu.async_copy(table.at[sidx.at[b]], rows[b], sems[b])

      def quad(jj, _):
        j0 = jj * NBUF
        for b in range(NBUF):
          j = j0 + b
          pltpu.make_async_copy(table.at[sidx.at[j]], rows[b], sems[b]).wait()
          nb = (b + NBUF - 1) % NBUF

          @pl.when(j + NBUF - 1 < K)
          def _start_next():
            pltpu.async_copy(table.at[sidx.at[j + NBUF - 1]], rows[nb], sems[nb])

          pltpu.sync_copy(rows[b], acc.at[didx.at[j]], add=True)
        return _

      lax.fori_loop(0, K // NBUF, quad, 0)
      plsc.subcore_barrier()
      pltpu.sync_copy(acc.at[pl.ds(base, RPT)], out.at[r, cid, pl.ds(base, RPT)])
      if r < nrel - 1:
        zero_rows()
        zero_slice()
      plsc.subcore_barrier()

  return pl.kernel(
      body,
      out_type=jax.ShapeDtypeStruct((nrel, NC, NNODE, W), jnp.float32),
      mesh=mesh,
      compiler_params=pltpu.CompilerParams(use_tc_tiling_on_sc=False),
      scratch_types=(
          [pltpu.VMEM_SHARED((NNODE, W), jnp.float32)]
          + [pltpu.VMEM((CHUNK, W), jnp.float32) for _ in range(NBUF)]
          + [pltpu.VMEM((K, CHUNK), jnp.int32) for _ in range(2)]
          + [pltpu.SemaphoreType.DMA for _ in range(NBUF)]
      ),
  )


BR = 2000
NB = NNODE // BR
SLAB = 10240
NZS = SLAB - NNODE
TBLK = 2048


def _row_mask(bdim):
  rows = pl.program_id(1) * TBLK + jax.lax.broadcasted_iota(jnp.int32, (TBLK, 1), 0)
  return rows < bdim


def _tc0_body(x_ref, out_ref):
  mask = _row_mask(NNODE)
  feat = jnp.where(mask, x_ref[0], 0.0)
  onescol = jnp.where(mask, jnp.ones((TBLK, 1), jnp.float32), 0.0)
  out_ref[...] = jnp.concatenate(
      [feat, onescol, jnp.zeros((TBLK, W - D - 1), jnp.float32)], axis=1)[None]


def _tc1_body(p_ref, wl_ref, xd_ref, wr_ref, bl_ref, out_ref):
  mask = _row_mask(NNODE)
  s = p_ref[:, 0] + p_ref[:, 1]
  acc = jnp.zeros((TBLK, D), jnp.float32)
  for r in range(3):
    deg = jnp.maximum(s[r, :, D:D + 1], 1.0)
    mean = s[r, :, :D] / deg
    acc = acc + jnp.dot(mean, wl_ref[r], preferred_element_type=jnp.float32)
  t = acc * (1.0 / 3.0) + jnp.dot(xd_ref[0], wr_ref[0],
                                  preferred_element_type=jnp.float32)
  t = t + bl_ref[pl.program_id(0)][None, :]
  h = jnp.where(mask, jnp.maximum(t, 0.0), 0.0)
  onescol = jnp.where(mask, jnp.ones((TBLK, 1), jnp.float32), 0.0)
  out_ref[...] = jnp.concatenate(
      [h, onescol, jnp.zeros((TBLK, W - D - 1), jnp.float32)], axis=1)[None]


def _tc2_body(p_ref, h_ref, wl_ref, wr_ref, bl_ref, wfc_ref, bfc_ref, out_ref):
  s = p_ref[:, 0] + p_ref[:, 1]
  acc = jnp.zeros((BR, D), jnp.float32)
  for r in range(3):
    deg = jnp.maximum(s[r, :, D:D + 1], 1.0)
    mean = s[r, :, :D] / deg
    acc = acc + jnp.dot(mean, wl_ref[r], preferred_element_type=jnp.float32)
  t = acc * (1.0 / 3.0) + jnp.dot(h_ref[0, :, :D], wr_ref[...],
                                  preferred_element_type=jnp.float32)
  t = t + bl_ref[0][None, :]
  h2 = jnp.maximum(t, 0.0)
  out_ref[...] = jnp.dot(h2, wfc_ref[...],
                         preferred_element_type=jnp.float32) + bfc_ref[0][None, :]


@jax.jit
def kernel(x_i, x_s, x_p, x_b, edges, params):
  x = {"i": x_i, "s": x_s, "p": x_p, "b": x_b}
  p1, p2 = params["conv1"], params["conv2"]

  npad = EPAD - 320000
  ar = jnp.arange(npad, dtype=jnp.int32)
  padz = (NNODE + ar % NZS)[None, :]
  pad_dst = jnp.broadcast_to(ar % NNODE, (9, npad))
  srcs9 = jnp.stack([edges[rel][0] for rel in REL1])
  dsts9 = jnp.stack([edges[rel][1] for rel in REL1])
  offs1 = jnp.array([TID[st] * SLAB for st in SRC1], jnp.int32)[:, None]
  src1 = jnp.concatenate([srcs9 + offs1, offs1 + padz],
                         axis=1).reshape(9, NT, K, CHUNK)
  dst1 = jnp.concatenate([dsts9, pad_dst], axis=1).reshape(9, NT, K, CHUNK)
  offs2 = (jnp.arange(3, dtype=jnp.int32) * SLAB)[:, None]
  src2 = jnp.concatenate([srcs9[:3] + offs2, offs2 + padz],
                         axis=1).reshape(3, NT, K, CHUNK)
  dst2 = dst1[:3]

  wl1 = jnp.stack([p1[rel]["Wl"] for rel in REL1])
  wr1 = jnp.stack([(p1[REL1[3 * g]]["Wr"] + p1[REL1[3 * g + 1]]["Wr"]
                    + p1[REL1[3 * g + 2]]["Wr"]) / 3.0 for g in range(3)])
  bl1 = jnp.stack([(p1[REL1[3 * g]]["bl"] + p1[REL1[3 * g + 1]]["bl"]
                    + p1[REL1[3 * g + 2]]["bl"]) / 3.0 for g in range(3)])
  xd = jnp.stack([x[d] for d in GRP_DST])
  wl2 = jnp.stack([p2[rel]["Wl"] for rel in REL2])
  wr2 = sum(p2[rel]["Wr"] for rel in REL2) / 3.0
  bl2 = (sum(p2[rel]["bl"] for rel in REL2) / 3.0)[None]
  wfc = params["fc"]["W"]
  bfc = params["fc"]["b"][None]

  xs = jnp.stack([x_i, x_s, x_p, x_b])
  table1 = pl.pallas_call(
      _tc0_body,
      grid=(4, SLAB // TBLK),
      in_specs=[pl.BlockSpec((1, TBLK, D), lambda t, b: (t, b, 0))],
      out_specs=pl.BlockSpec((1, TBLK, W), lambda t, b: (t, b, 0)),
      out_shape=jax.ShapeDtypeStruct((4, SLAB, W), jnp.float32),
  )(xs).reshape(4 * SLAB, W)

  part1 = _sc_segment_sums(9)(table1, src1, dst1)

  h_ext = pl.pallas_call(
      _tc1_body,
      grid=(3, SLAB // TBLK),
      in_specs=[
          pl.BlockSpec((3, NC, TBLK, W), lambda g, b: (g, 0, b, 0)),
          pl.BlockSpec((3, D, D), lambda g, b: (g, 0, 0)),
          pl.BlockSpec((1, TBLK, D), lambda g, b: (g, b, 0)),
          pl.BlockSpec((1, D, D), lambda g, b: (g, 0, 0)),
          pl.BlockSpec((3, D), lambda g, b: (0, 0)),
      ],
      out_specs=pl.BlockSpec((1, TBLK, W), lambda g, b: (2 - g, b, 0)),
      out_shape=jax.ShapeDtypeStruct((3, SLAB, W), jnp.float32),
  )(part1, wl1, xd, wr1, bl1)

  table2 = h_ext.reshape(3 * SLAB, W)
  part2 = _sc_segment_sums(3)(table2, src2, dst2)

  out = pl.pallas_call(
      _tc2_body,
      grid=(NB,),
      in_specs=[
          pl.BlockSpec((3, NC, BR, W), lambda b: (0, 0, b, 0)),
          pl.BlockSpec((1, BR, W), lambda b: (2, b, 0)),
          pl.BlockSpec((3, D, D), lambda b: (0, 0, 0)),
          pl.BlockSpec((D, D), lambda b: (0, 0)),
          pl.BlockSpec((1, D), lambda b: (0, 0)),
          pl.BlockSpec((D, 64), lambda b: (0, 0)),
          pl.BlockSpec((1, 64), lambda b: (0, 0)),
      ],
      out_specs=pl.BlockSpec((BR, 64), lambda b: (b, 0)),
      out_shape=jax.ShapeDtypeStruct((NNODE, 64), jnp.float32),
  )(part2, h_ext, wl2, wr2, bl2, wfc, bfc)
  return out


# speedup vs baseline: 1.0576x; 1.0576x over previous
import functools

import jax
import jax.numpy as jnp
from jax import lax
from jax.experimental import pallas as pl
from jax.experimental.pallas import tpu as pltpu
from jax.experimental.pallas import tpu_sc as plsc

NNODE = 10000
D = 128
W = 144
NC, NS = 2, 16
NT = NC * NS
CHUNK = 32
K = 320
NBUF = 5
QCH = 80
EPAD = NT * K * CHUNK
NZROW = 16
RPT = NNODE // NS

REL1 = ["p2b", "s2b", "b2b", "i2s", "p2s", "b2s", "b2p", "p2p", "s2p"]
SRC1 = ["p", "s", "b", "i", "p", "b", "b", "p", "s"]
GRP_DST = ["b", "s", "p"]
TID = {"i": 0, "s": 1, "p": 2, "b": 3}
REL2 = ["p2b", "s2b", "b2b"]


def _sc_segment_sums(nrel):
  mesh = plsc.VectorSubcoreMesh(
      core_axis_name="c", subcore_axis_name="s", num_cores=NC, num_subcores=NS)

  def body(table, src, dst, out, acc, r0, r1, r2, r3, r4,
           si0, si1, di0, di1, *sems):
    rows = [r0, r1, r2, r3, r4]
    sidx = [si0, si1]
    didx = [di0, di1]
    semi = sems[NBUF]
    cid = lax.axis_index("c")
    sid = lax.axis_index("s")
    wid = cid * NS + sid
    base = sid * RPT

    def zero_rows():
      zvec = jnp.zeros((16,), jnp.float32)

      def zstore(i, _):
        rr = i // (W // 16)
        cc = (i % (W // 16)) * 16
        rows[0][rr, pl.ds(cc, 16)] = zvec
        rows[1][rr, pl.ds(cc, 16)] = zvec
        return _

      lax.fori_loop(0, CHUNK * (W // 16), zstore, 0)

    def zero_slice():
      def zcopy(t, _):
        pltpu.sync_copy(rows[0], acc.at[pl.ds(base + 2 * t * CHUNK, CHUNK)])
        pltpu.sync_copy(rows[1], acc.at[pl.ds(base + (2 * t + 1) * CHUNK, CHUNK)])
        return _

      lax.fori_loop(0, RPT // (2 * CHUNK), zcopy, 0)
      rem = RPT % (2 * CHUNK)
      if rem:
        pltpu.sync_copy(rows[0].at[pl.ds(0, rem)],
                        acc.at[pl.ds(base + (RPT // (2 * CHUNK)) * 2 * CHUNK, rem)])

    zero_rows()
    zero_slice()
    plsc.subcore_barrier()

    for r in range(nrel):
      pltpu.sync_copy(src.at[r, wid, pl.ds(0, QCH)], sidx[0])
      pltpu.sync_copy(dst.at[r, wid, pl.ds(0, QCH)], didx[0])
      for q in range(K // QCH):
        p = q % 2
        sq, dq = sidx[p], didx[p]
        if q < K // QCH - 1:
          pltpu.async_copy(src.at[r, wid, pl.ds((q + 1) * QCH, QCH)],
                           sidx[1 - p], semi)
          pltpu.async_copy(dst.at[r, wid, pl.ds((q + 1) * QCH, QCH)],
                           didx[1 - p], semi)
        for b in range(NBUF - 1):
          pltpu.async_copy(table.at[sq.at[b]], rows[b], sems[b])

        def ring(i, _):
          c0 = i * NBUF
          for b in range(NBUF):
            c = c0 + b
            pltpu.make_async_copy(table.at[sq.at[c]], rows[b], sems[b]).wait()
            nb = (b + NBUF - 1) % NBUF

            @pl.when(c + NBUF - 1 < QCH)
            def _start_next():
              pltpu.async_copy(table.at[sq.at[c + NBUF - 1]], rows[nb], sems[nb])

            pltpu.sync_copy(rows[b], acc.at[dq.at[c]], add=True)
          return _

        lax.fori_loop(0, QCH // NBUF, ring, 0)
        if q < K // QCH - 1:
          pltpu.make_async_copy(src.at[r, wid, pl.ds((q + 1) * QCH, QCH)],
                                sidx[1 - p], semi).wait()
          pltpu.make_async_copy(dst.at[r, wid, pl.ds((q + 1) * QCH, QCH)],
                                didx[1 - p], semi).wait()
      plsc.subcore_barrier()
      pltpu.sync_copy(acc.at[pl.ds(base, RPT)], out.at[r, cid, pl.ds(base, RPT)])
      if r < nrel - 1:
        zero_rows()
        zero_slice()
      plsc.subcore_barrier()

  return pl.kernel(
      body,
      out_type=jax.ShapeDtypeStruct((nrel, NC, NNODE, W), jnp.float32),
      mesh=mesh,
      compiler_params=pltpu.CompilerParams(use_tc_tiling_on_sc=False),
      scratch_types=(
          [pltpu.VMEM_SHARED((NNODE, W), jnp.float32)]
          + [pltpu.VMEM((CHUNK, W), jnp.float32) for _ in range(NBUF)]
          + [pltpu.VMEM((QCH, CHUNK), jnp.int32) for _ in range(4)]
          + [pltpu.SemaphoreType.DMA for _ in range(NBUF + 1)]
      ),
  )


BR = 2000
NB = NNODE // BR
SLAB = 10240
NZS = SLAB - NNODE
TBLK = 2048


def _row_mask(bdim):
  rows = pl.program_id(1) * TBLK + jax.lax.broadcasted_iota(jnp.int32, (TBLK, 1), 0)
  return rows < bdim


def _tc0_body(x_ref, out_ref):
  mask = _row_mask(NNODE)
  feat = jnp.where(mask, x_ref[0], 0.0)
  onescol = jnp.where(mask, jnp.ones((TBLK, 1), jnp.float32), 0.0)
  out_ref[...] = jnp.concatenate(
      [feat, onescol, jnp.zeros((TBLK, W - D - 1), jnp.float32)], axis=1)[None]


def _tc1_body(p_ref, wl_ref, xd_ref, wr_ref, bl_ref, out_ref):
  mask = _row_mask(NNODE)
  s = p_ref[:, 0] + p_ref[:, 1]
  acc = jnp.zeros((TBLK, D), jnp.float32)
  for r in range(3):
    deg = jnp.maximum(s[r, :, D:D + 1], 1.0)
    mean = s[r, :, :D] / deg
    acc = acc + jnp.dot(mean, wl_ref[r], preferred_element_type=jnp.float32)
  t = acc * (1.0 / 3.0) + jnp.dot(xd_ref[0], wr_ref[0],
                                  preferred_element_type=jnp.float32)
  t = t + bl_ref[pl.program_id(0)][None, :]
  h = jnp.where(mask, jnp.maximum(t, 0.0), 0.0)
  onescol = jnp.where(mask, jnp.ones((TBLK, 1), jnp.float32), 0.0)
  out_ref[...] = jnp.concatenate(
      [h, onescol, jnp.zeros((TBLK, W - D - 1), jnp.float32)], axis=1)[None]


def _tc2_body(p_ref, h_ref, wl_ref, wr_ref, bl_ref, wfc_ref, bfc_ref, out_ref):
  s = p_ref[:, 0] + p_ref[:, 1]
  acc = jnp.zeros((BR, D), jnp.float32)
  for r in range(3):
    deg = jnp.maximum(s[r, :, D:D + 1], 1.0)
    mean = s[r, :, :D] / deg
    acc = acc + jnp.dot(mean, wl_ref[r], preferred_element_type=jnp.float32)
  t = acc * (1.0 / 3.0) + jnp.dot(h_ref[0, :, :D], wr_ref[...],
                                  preferred_element_type=jnp.float32)
  t = t + bl_ref[0][None, :]
  h2 = jnp.maximum(t, 0.0)
  out_ref[...] = jnp.dot(h2, wfc_ref[...],
                         preferred_element_type=jnp.float32) + bfc_ref[0][None, :]


@jax.jit
def kernel(x_i, x_s, x_p, x_b, edges, params):
  x = {"i": x_i, "s": x_s, "p": x_p, "b": x_b}
  p1, p2 = params["conv1"], params["conv2"]

  npad = EPAD - 320000
  ar = jnp.arange(npad, dtype=jnp.int32)
  padz = (NNODE + ar % NZS)[None, :]
  pad_dst = jnp.broadcast_to(ar % NNODE, (9, npad))
  srcs9 = jnp.stack([edges[rel][0] for rel in REL1])
  dsts9 = jnp.stack([edges[rel][1] for rel in REL1])
  offs1 = jnp.array([TID[st] * SLAB for st in SRC1], jnp.int32)[:, None]
  src1 = jnp.concatenate([srcs9 + offs1, offs1 + padz],
                         axis=1).reshape(9, NT, K, CHUNK)
  dst1 = jnp.concatenate([dsts9, pad_dst], axis=1).reshape(9, NT, K, CHUNK)
  offs2 = (jnp.arange(3, dtype=jnp.int32) * SLAB)[:, None]
  src2 = jnp.concatenate([srcs9[:3] + offs2, offs2 + padz],
                         axis=1).reshape(3, NT, K, CHUNK)
  dst2 = dst1[:3]

  wl1 = jnp.stack([p1[rel]["Wl"] for rel in REL1])
  wr1 = jnp.stack([(p1[REL1[3 * g]]["Wr"] + p1[REL1[3 * g + 1]]["Wr"]
                    + p1[REL1[3 * g + 2]]["Wr"]) / 3.0 for g in range(3)])
  bl1 = jnp.stack([(p1[REL1[3 * g]]["bl"] + p1[REL1[3 * g + 1]]["bl"]
                    + p1[REL1[3 * g + 2]]["bl"]) / 3.0 for g in range(3)])
  xd = jnp.stack([x[d] for d in GRP_DST])
  wl2 = jnp.stack([p2[rel]["Wl"] for rel in REL2])
  wr2 = sum(p2[rel]["Wr"] for rel in REL2) / 3.0
  bl2 = (sum(p2[rel]["bl"] for rel in REL2) / 3.0)[None]
  wfc = params["fc"]["W"]
  bfc = params["fc"]["b"][None]

  xs = jnp.stack([x_i, x_s, x_p, x_b])
  table1 = pl.pallas_call(
      _tc0_body,
      grid=(4, SLAB // TBLK),
      in_specs=[pl.BlockSpec((1, TBLK, D), lambda t, b: (t, b, 0))],
      out_specs=pl.BlockSpec((1, TBLK, W), lambda t, b: (t, b, 0)),
      out_shape=jax.ShapeDtypeStruct((4, SLAB, W), jnp.float32),
  )(xs).reshape(4 * SLAB, W)

  part1 = _sc_segment_sums(9)(table1, src1, dst1)

  h_ext = pl.pallas_call(
      _tc1_body,
      grid=(3, SLAB // TBLK),
      in_specs=[
          pl.BlockSpec((3, NC, TBLK, W), lambda g, b: (g, 0, b, 0)),
          pl.BlockSpec((3, D, D), lambda g, b: (g, 0, 0)),
          pl.BlockSpec((1, TBLK, D), lambda g, b: (g, b, 0)),
          pl.BlockSpec((1, D, D), lambda g, b: (g, 0, 0)),
          pl.BlockSpec((3, D), lambda g, b: (0, 0)),
      ],
      out_specs=pl.BlockSpec((1, TBLK, W), lambda g, b: (2 - g, b, 0)),
      out_shape=jax.ShapeDtypeStruct((3, SLAB, W), jnp.float32),
  )(part1, wl1, xd, wr1, bl1)

  table2 = h_ext.reshape(3 * SLAB, W)
  part2 = _sc_segment_sums(3)(table2, src2, dst2)

  out = pl.pallas_call(
      _tc2_body,
      grid=(NB,),
      in_specs=[
          pl.BlockSpec((3, NC, BR, W), lambda b: (0, 0, b, 0)),
          pl.BlockSpec((1, BR, W), lambda b: (2, b, 0)),
          pl.BlockSpec((3, D, D), lambda b: (0, 0, 0)),
          pl.BlockSpec((D, D), lambda b: (0, 0)),
          pl.BlockSpec((1, D), lambda b: (0, 0)),
          pl.BlockSpec((D, 64), lambda b: (0, 0)),
          pl.BlockSpec((1, 64), lambda b: (0, 0)),
      ],
      out_specs=pl.BlockSpec((BR, 64), lambda b: (b, 0)),
      out_shape=jax.ShapeDtypeStruct((NNODE, 64), jnp.float32),
  )(part2, h_ext, wl2, wr2, bl2, wfc, bfc)
  return out


# speedup vs baseline: 1.1783x; 1.1140x over previous
import jax
import jax.numpy as jnp
from jax import lax
from jax.experimental import pallas as pl
from jax.experimental.pallas import tpu as pltpu
from jax.experimental.pallas import tpu_sc as plsc

NNODE = 10000
D = 128
DW = 16
NC, NS = 2, 16
NT = NC * NS
CHUNK = 32
K = 320
NBUF = 5
QCH = 80
EPAD = NT * K * CHUNK
RPT = NNODE // NS
SLAB = 10240
NZS = SLAB - NNODE
TBLK = 2048
BR = 2000
NB = NNODE // BR

REL1 = ["p2b", "s2b", "b2b", "i2s", "p2s", "b2s", "b2p", "p2p", "s2p"]
SRC1 = ["p", "s", "b", "i", "p", "b", "b", "p", "s"]
GRP_DST = ["b", "s", "p"]
TID = {"i": 0, "s": 1, "p": 2, "b": 3}
REL2 = ["p2b", "s2b", "b2b"]


def _sc_segment_sums(nrel, with_deg):
  mesh = plsc.VectorSubcoreMesh(
      core_axis_name="c", subcore_axis_name="s", num_cores=NC, num_subcores=NS)

  def body(*refs):
    if with_deg:
      (table, src, dst, out, dout, acc, dacc, r0, r1, r2, r3, r4,
       ones, zb16, si0, si1, di0, di1, *sems) = refs
    else:
      (table, src, dst, out, acc, r0, r1, r2, r3, r4,
       si0, si1, di0, di1, *sems) = refs
    rows = [r0, r1, r2, r3, r4]
    sidx = [si0, si1]
    didx = [di0, di1]
    semi = sems[NBUF]
    cid = lax.axis_index("c")
    sid = lax.axis_index("s")
    wid = cid * NS + sid
    base = sid * RPT
    zvec = jnp.zeros((16,), jnp.float32)

    def zero_rows():
      def zstore(i, _):
        rr = i // (D // 16)
        cc = (i % (D // 16)) * 16
        rows[0][rr, pl.ds(cc, 16)] = zvec
        return _

      lax.fori_loop(0, CHUNK * (D // 16), zstore, 0)

    def zero_slice():
      def zcopy(t, _):
        pltpu.sync_copy(rows[0], acc.at[pl.ds(base + t * CHUNK, CHUNK)])
        return _

      lax.fori_loop(0, RPT // CHUNK, zcopy, 0)
      pltpu.sync_copy(rows[0].at[pl.ds(0, RPT % CHUNK)],
                      acc.at[pl.ds(base + (RPT // CHUNK) * CHUNK, RPT % CHUNK)])

    def zero_deg():
      def zdcopy(t, _):
        pltpu.sync_copy(zb16, dacc.at[pl.ds(base + t * (RPT // 5), RPT // 5)])
        return _

      lax.fori_loop(0, 5, zdcopy, 0)

    zero_rows()
    zero_slice()
    if with_deg:
      def fill(i, _):
        zb16[i // 1, pl.ds(0, 16)] = zvec
        return _

      lax.fori_loop(0, RPT // 5, fill, 0)

      def fill1(i, _):
        ones[i // 1, pl.ds(0, 16)] = jnp.ones((16,), jnp.float32)
        return _

      lax.fori_loop(0, CHUNK, fill1, 0)
      zero_deg()
    plsc.subcore_barrier()

    def rel_body(r, _r):
      pltpu.sync_copy(src.at[r, wid, pl.ds(0, QCH)], sidx[0])
      pltpu.sync_copy(dst.at[r, wid, pl.ds(0, QCH)], didx[0])
      for q in range(K // QCH):
        p = q % 2
        sq, dq = sidx[p], didx[p]
        if q < K // QCH - 1:
          pltpu.async_copy(src.at[r, wid, pl.ds((q + 1) * QCH, QCH)],
                           sidx[1 - p], semi)
          pltpu.async_copy(dst.at[r, wid, pl.ds((q + 1) * QCH, QCH)],
                           didx[1 - p], semi)
        for b in range(NBUF - 1):
          pltpu.async_copy(table.at[sq.at[b]], rows[b], sems[b])

        def ring(i, _):
          c0 = i * NBUF
          for b in range(NBUF):
            c = c0 + b
            pltpu.make_async_copy(table.at[sq.at[c]], rows[b], sems[b]).wait()
            nb = (b + NBUF - 1) % NBUF

            @pl.when(c + NBUF - 1 < QCH)
            def _start_next():
              pltpu.async_copy(table.at[sq.at[c + NBUF - 1]], rows[nb], sems[nb])

            pltpu.sync_copy(rows[b], acc.at[dq.at[c]], add=True)
            if with_deg:
              pltpu.sync_copy(ones, dacc.at[dq.at[c]], add=True)
          return _

        lax.fori_loop(0, QCH // NBUF, ring, 0)
        if q < K // QCH - 1:
          pltpu.make_async_copy(src.at[r, wid, pl.ds((q + 1) * QCH, QCH)],
                                sidx[1 - p], semi).wait()
          pltpu.make_async_copy(dst.at[r, wid, pl.ds((q + 1) * QCH, QCH)],
                                didx[1 - p], semi).wait()
      plsc.subcore_barrier()
      pltpu.sync_copy(acc.at[pl.ds(base, RPT)], out.at[r, cid, pl.ds(base, RPT)])
      if with_deg:
        pltpu.sync_copy(dacc.at[pl.ds(base, RPT)],
                        dout.at[r, cid, pl.ds(base, RPT)])
      zero_rows()
      zero_slice()
      if with_deg:
        zero_deg()
      plsc.subcore_barrier()
      return _r

    lax.fori_loop(0, nrel, rel_body, 0)

  if with_deg:
    out_type = [jax.ShapeDtypeStruct((nrel, NC, NNODE, D), jnp.float32),
                jax.ShapeDtypeStruct((nrel, NC, NNODE, DW), jnp.float32)]
    extra = [pltpu.VMEM_SHARED((NNODE, DW), jnp.float32)]
    extra2 = [pltpu.VMEM((CHUNK, DW), jnp.float32),
              pltpu.VMEM((RPT // 5, DW), jnp.float32)]
  else:
    out_type = jax.ShapeDtypeStruct((nrel, NC, NNODE, D), jnp.float32)
    extra = []
    extra2 = []

  return pl.kernel(
      body,
      out_type=out_type,
      mesh=mesh,
      compiler_params=pltpu.CompilerParams(use_tc_tiling_on_sc=False),
      scratch_types=(
          [pltpu.VMEM_SHARED((NNODE, D), jnp.float32)] + extra
          + [pltpu.VMEM((CHUNK, D), jnp.float32) for _ in range(NBUF)]
          + extra2
          + [pltpu.VMEM((QCH, CHUNK), jnp.int32) for _ in range(4)]
          + [pltpu.SemaphoreType.DMA for _ in range(NBUF + 1)]
      ),
  )


NPAD = EPAD - 320000


def _rows(bdim, pid_axis):
  return (pl.program_id(pid_axis) * bdim
          + jax.lax.broadcasted_iota(jnp.int32, (bdim, 1), 0))


def _row_mask():
  return _rows(TBLK, 1) < NNODE


def _tc0_body(x_ref, out_ref):
  out_ref[...] = jnp.where(_row_mask(), x_ref[0], 0.0)[None]


def _tc1_body(p_ref, dg_ref, wl_ref, xd_ref, wr_ref, bl_ref, out_ref):
  mask = _row_mask()
  corr = jnp.where(_rows(TBLK, 1) < NPAD, 1.0, 0.0)
  s = p_ref[:, 0] + p_ref[:, 1]
  dg = dg_ref[:, 0] + dg_ref[:, 1]
  acc = jnp.zeros((TBLK, D), jnp.float32)
  for r in range(3):
    mean = s[r] / jnp.maximum(dg[r, :, 0:1] - corr, 1.0)
    acc = acc + jnp.dot(mean, wl_ref[r], preferred_element_type=jnp.float32)
  t = acc * (1.0 / 3.0) + jnp.dot(xd_ref[0], wr_ref[0],
                                  preferred_element_type=jnp.float32)
  t = t + bl_ref[pl.program_id(0)][None, :]
  out_ref[...] = jnp.where(mask, jnp.maximum(t, 0.0), 0.0)[None]


def _tc2_body(p_ref, dg_ref, h_ref, wl_ref, wr_ref, bl_ref, wfc_ref, bfc_ref,
              out_ref):
  corr = jnp.where(_rows(BR, 0) < NPAD, 1.0, 0.0)
  s = p_ref[:, 0] + p_ref[:, 1]
  dg = dg_ref[:, 0] + dg_ref[:, 1]
  acc = jnp.zeros((BR, D), jnp.float32)
  for r in range(3):
    mean = s[r] / jnp.maximum(dg[r, :, 0:1] - corr, 1.0)
    acc = acc + jnp.dot(mean, wl_ref[r], preferred_element_type=jnp.float32)
  t = acc * (1.0 / 3.0) + jnp.dot(h_ref[0], wr_ref[...],
                                  preferred_element_type=jnp.float32)
  t = t + bl_ref[0][None, :]
  h2 = jnp.maximum(t, 0.0)
  out_ref[...] = jnp.dot(h2, wfc_ref[...],
                         preferred_element_type=jnp.float32) + bfc_ref[0][None, :]


@jax.jit
def kernel(x_i, x_s, x_p, x_b, edges, params):
  x = {"i": x_i, "s": x_s, "p": x_p, "b": x_b}
  p1, p2 = params["conv1"], params["conv2"]

  npad = EPAD - 320000
  ar = jnp.arange(npad, dtype=jnp.int32)
  padz = (NNODE + ar % NZS)[None, :]
  pad_dst = jnp.broadcast_to(ar % NNODE, (9, npad))
  srcs9 = jnp.stack([edges[rel][0] for rel in REL1])
  dsts9 = jnp.stack([edges[rel][1] for rel in REL1])
  offs1 = jnp.array([TID[st] * SLAB for st in SRC1], jnp.int32)[:, None]
  src1 = jnp.concatenate([srcs9 + offs1, offs1 + padz],
                         axis=1).reshape(9, NT, K, CHUNK)
  dst1 = jnp.concatenate([dsts9, pad_dst], axis=1).reshape(9, NT, K, CHUNK)
  offs2 = (jnp.arange(3, dtype=jnp.int32) * SLAB)[:, None]
  src2 = jnp.concatenate([srcs9[:3] + offs2, offs2 + padz],
                         axis=1).reshape(3, NT, K, CHUNK)
  dst2 = dst1[:3]

  wl1 = jnp.stack([p1[rel]["Wl"] for rel in REL1])
  wr1 = jnp.stack([(p1[REL1[3 * g]]["Wr"] + p1[REL1[3 * g + 1]]["Wr"]
                    + p1[REL1[3 * g + 2]]["Wr"]) / 3.0 for g in range(3)])
  bl1 = jnp.stack([(p1[REL1[3 * g]]["bl"] + p1[REL1[3 * g + 1]]["bl"]
                    + p1[REL1[3 * g + 2]]["bl"]) / 3.0 for g in range(3)])
  xd = jnp.stack([x[d] for d in GRP_DST])
  wl2 = jnp.stack([p2[rel]["Wl"] for rel in REL2])
  wr2 = sum(p2[rel]["Wr"] for rel in REL2) / 3.0
  bl2 = (sum(p2[rel]["bl"] for rel in REL2) / 3.0)[None]
  wfc = params["fc"]["W"]
  bfc = params["fc"]["b"][None]

  xs = jnp.stack([x_i, x_s, x_p, x_b])
  table1 = pl.pallas_call(
      _tc0_body,
      grid=(4, SLAB // TBLK),
      in_specs=[pl.BlockSpec((1, TBLK, D), lambda t, b: (t, b, 0))],
      out_specs=pl.BlockSpec((1, TBLK, D), lambda t, b: (t, b, 0)),
      out_shape=jax.ShapeDtypeStruct((4, SLAB, D), jnp.float32),
  )(xs).reshape(4 * SLAB, D)

  part1, deg1 = _sc_segment_sums(9, True)(table1, src1, dst1)

  h1 = pl.pallas_call(
      _tc1_body,
      grid=(3, SLAB // TBLK),
      in_specs=[
          pl.BlockSpec((3, NC, TBLK, D), lambda g, b: (g, 0, b, 0)),
          pl.BlockSpec((3, NC, TBLK, DW), lambda g, b: (g, 0, b, 0)),
          pl.BlockSpec((3, D, D), lambda g, b: (g, 0, 0)),
          pl.BlockSpec((1, TBLK, D), lambda g, b: (g, b, 0)),
          pl.BlockSpec((1, D, D), lambda g, b: (g, 0, 0)),
          pl.BlockSpec((3, D), lambda g, b: (0, 0)),
      ],
      out_specs=pl.BlockSpec((1, TBLK, D), lambda g, b: (2 - g, b, 0)),
      out_shape=jax.ShapeDtypeStruct((3, SLAB, D), jnp.float32),
  )(part1, deg1, wl1, xd, wr1, bl1)

  table2 = h1.reshape(3 * SLAB, D)
  part2 = _sc_segment_sums(3, False)(table2, src2, dst2)

  out = pl.pallas_call(
      _tc2_body,
      grid=(NB,),
      in_specs=[
          pl.BlockSpec((3, NC, BR, D), lambda b: (0, 0, b, 0)),
          pl.BlockSpec((3, NC, BR, DW), lambda b: (0, 0, b, 0)),
          pl.BlockSpec((1, BR, D), lambda b: (2, b, 0)),
          pl.BlockSpec((3, D, D), lambda b: (0, 0, 0)),
          pl.BlockSpec((D, D), lambda b: (0, 0)),
          pl.BlockSpec((1, D), lambda b: (0, 0)),
          pl.BlockSpec((D, 64), lambda b: (0, 0)),
          pl.BlockSpec((1, 64), lambda b: (0, 0)),
      ],
      out_specs=pl.BlockSpec((BR, 64), lambda b: (b, 0)),
      out_shape=jax.ShapeDtypeStruct((NNODE, 64), jnp.float32),
  )(part2, deg1, h1, wl2, wr2, bl2, wfc, bfc)
  return out
